# baseline (speedup 1.0000x reference)
import jax
import jax.numpy as jnp
from jax.experimental import pallas as pl
from jax.experimental.pallas import tpu as pltpu

_N, _P, _B, _D, _M = 100000, 256, 4096, 128, 32
_BN = 5000
_G = _N // _BN


def _fused_body(meta_ref, card_ref, w_ref, b_ref, pools_ref, out_ref, acc_ref):
    i = pl.program_id(0)

    @pl.when(i == 0)
    def _init():
        acc_ref[...] = jnp.zeros_like(acc_ref)

    pre = jnp.dot(card_ref[...], w_ref[...],
                  preferred_element_type=jnp.float32) + b_ref[...]
    transformed = pre * jax.nn.sigmoid(pre)
    acc_ref[...] += jax.lax.dot_general(
        meta_ref[...].astype(jnp.bfloat16), transformed.astype(jnp.bfloat16),
        (((0,), (0,)), ((), ())),
        preferred_element_type=jnp.float32)

    @pl.when(i == _G - 1)
    def _finish():
        out_ref[...] = jnp.dot(pools_ref[...], acc_ref[...],
                               preferred_element_type=jnp.float32)


def kernel(batch_pools, metapath, card_embeddings, W, b_dense):
    b2 = b_dense.reshape(1, _M)
    return pl.pallas_call(
        _fused_body,
        grid=(_G,),
        in_specs=[
            pl.BlockSpec((_BN, _P), lambda i: (i, 0)),
            pl.BlockSpec((_BN, _D), lambda i: (i, 0)),
            pl.BlockSpec((_D, _M), lambda i: (0, 0)),
            pl.BlockSpec((1, _M), lambda i: (0, 0)),
            pl.BlockSpec((_B, _P), lambda i: (0, 0)),
        ],
        out_specs=pl.BlockSpec((_B, _M), lambda i: (0, 0)),
        out_shape=jax.ShapeDtypeStruct((_B, _M), jnp.float32),
        scratch_shapes=[pltpu.VMEM((_P, _M), jnp.float32)],
    )(metapath, card_embeddings, W, b2, batch_pools)


# speedup vs baseline: 1.0766x; 1.0766x over previous
import jax
import jax.numpy as jnp
from jax.experimental import pallas as pl
from jax.experimental.pallas import tpu as pltpu

_N, _P, _B, _D, _M = 100000, 256, 4096, 128, 32
_BN = 10000
_G = _N // _BN


def _fused_body(meta_ref, card_ref, w_ref, b_ref, pools_ref, out_ref, acc_ref):
    i = pl.program_id(0)

    @pl.when(i == 0)
    def _init():
        acc_ref[...] = jnp.zeros_like(acc_ref)

    pre = jnp.dot(card_ref[...], w_ref[...],
                  preferred_element_type=jnp.float32) + b_ref[...]
    transformed = pre * jax.nn.sigmoid(pre)
    acc_ref[...] += jax.lax.dot_general(
        meta_ref[...].astype(jnp.bfloat16), transformed.astype(jnp.bfloat16),
        (((0,), (0,)), ((), ())),
        preferred_element_type=jnp.float32)

    @pl.when(i == _G - 1)
    def _finish():
        out_ref[...] = jnp.dot(pools_ref[...], acc_ref[...],
                               preferred_element_type=jnp.float32)


def kernel(batch_pools, metapath, card_embeddings, W, b_dense):
    b2 = b_dense.reshape(1, _M)
    return pl.pallas_call(
        _fused_body,
        grid=(_G,),
        in_specs=[
            pl.BlockSpec((_BN, _P), lambda i: (i, 0)),
            pl.BlockSpec((_BN, _D), lambda i: (i, 0)),
            pl.BlockSpec((_D, _M), lambda i: (0, 0)),
            pl.BlockSpec((1, _M), lambda i: (0, 0)),
            pl.BlockSpec((_B, _P), lambda i: (0, 0)),
        ],
        out_specs=pl.BlockSpec((_B, _M), lambda i: (0, 0)),
        out_shape=jax.ShapeDtypeStruct((_B, _M), jnp.float32),
        scratch_shapes=[pltpu.VMEM((_P, _M), jnp.float32)],
    )(metapath, card_embeddings, W, b2, batch_pools)
